# baseline (speedup 1.0000x reference)
import functools

import jax
import jax.numpy as jnp
from jax import lax
from jax.experimental import pallas as pl
from jax.experimental.pallas import tpu as pltpu
from jax.experimental.pallas import tpu_sc as plsc

B = 4096
N = 20
S = 20
E = 32

NC, NS = 2, 16
NW = NC * NS
BPW = B // NW
CB = 4
CI = CB * N * S
CR = CB * S
GSZ = 80
NCHUNK = BPW // CB

_HI = jnp.uint32(0xFFFF0000)


def _sc_body(idx_hbm, table_hbm, out_hbm, idx_v, rows_v, out_v, sem):
    wid = lax.axis_index("s") * NC + lax.axis_index("c")
    wb = wid * BPW
    lanes = lax.iota(jnp.int32, 16)
    even = lanes * 2
    odd = even + 1

    def acc_body(r, carry):
        bb = r // S
        s = r - bb * S
        base = bb * (N * S) + s
        w = plsc.bitcast(rows_v[base, :], jnp.uint32)
        a = plsc.bitcast(w << 16, jnp.float32)
        b = plsc.bitcast(w & _HI, jnp.float32)
        for n in range(1, N):
            w = plsc.bitcast(rows_v[base + n * S, :], jnp.uint32)
            a = a + plsc.bitcast(w << 16, jnp.float32)
            b = b + plsc.bitcast(w & _HI, jnp.float32)
        bbv = jnp.full((16,), bb, jnp.int32)
        sv = jnp.full((16,), s, jnp.int32)
        plsc.store_scatter(out_v, [bbv, sv, even], a)
        plsc.store_scatter(out_v, [bbv, sv, odd], b)
        return carry

    def chunk_body(c, carry):
        b0 = wb + c * CB
        pltpu.sync_copy(idx_hbm.at[pl.ds(b0, CB), :], idx_v)
        descs = [
            pltpu.async_copy(
                table_hbm.at[idx_v.at[bb, pl.ds(g * GSZ, GSZ)]],
                rows_v.at[pl.ds(bb * (N * S) + g * GSZ, GSZ), :],
                sem,
            )
            for bb in range(CB)
            for g in range(N * S // GSZ)
        ]
        for d in descs:
            d.wait()
        lax.fori_loop(0, CR, acc_body, 0)
        pltpu.sync_copy(out_v, out_hbm.at[pl.ds(b0, CB), 0, :, :])
        return carry

    lax.fori_loop(0, NCHUNK, chunk_body, 0)


@functools.cache
def _sc_call():
    return functools.partial(
        pl.kernel,
        out_type=jax.ShapeDtypeStruct((B, 1, S, E), jnp.float32),
        mesh=plsc.VectorSubcoreMesh(
            core_axis_name="c", subcore_axis_name="s",
            num_cores=NC, num_subcores=NS,
        ),
        scratch_types=[
            pltpu.VMEM((CB, N * S), jnp.int32),
            pltpu.VMEM((CI, E), jnp.bfloat16),
            pltpu.VMEM((CB, S, E), jnp.float32),
            pltpu.SemaphoreType.DMA,
        ],
        compiler_params=pltpu.CompilerParams(
            use_tc_tiling_on_sc=False, needs_layout_passes=False
        ),
    )(_sc_body)


def kernel(x, table):
    return _sc_call()(x.reshape(B, N * S), table.astype(jnp.bfloat16))
